# baseline (speedup 1.0000x reference)
import functools

import jax
import jax.numpy as jnp
from jax import lax
from jax.experimental import pallas as pl
from jax.experimental.pallas import tpu as pltpu
from jax.experimental.pallas import tpu_sc as plsc

_LANES = 16
_CH = 128
_NSLOT = 4


@functools.cache
def _build(batch, seq, embed, vocab):
    info = plsc.get_sparse_core_info()
    nw = info.num_cores * info.num_subcores
    assert batch % nw == 0 and seq == _NSLOT * _CH and embed == 64
    rows_per_w = batch // nw
    nch = rows_per_w * _NSLOT
    st_n = seq // _CH

    mesh = plsc.VectorSubcoreMesh(core_axis_name="c", subcore_axis_name="s")

    @functools.partial(
        pl.kernel,
        out_type=jax.ShapeDtypeStruct((batch, 8, st_n, 8, _CH), jnp.float32),
        mesh=mesh,
        compiler_params=pltpu.CompilerParams(
            use_tc_tiling_on_sc=False, needs_layout_passes=False
        ),
        scratch_types=(
            [pltpu.MemorySpace.VMEM_SHARED((st_n, 8, 8, _CH), jnp.float32)]
            + [pltpu.VMEM((8, 8, _CH), jnp.float32)]
            + [pltpu.VMEM((_CH,), jnp.int32) for _ in range(_NSLOT)]
            + [pltpu.VMEM((_CH, 64), jnp.float32) for _ in range(_NSLOT)]
            + [pltpu.VMEM((8, 8, _CH), jnp.float32) for _ in range(_NSLOT)]
            + [pltpu.SemaphoreType.DMA for _ in range(4 * _NSLOT)]
        ),
    )
    def embed_kernel(ids_hbm, tok_hbm, pos_hbm, out_hbm, *scratch):
        pt_sh = scratch[0]
        pt_tmp = scratch[1]
        o = 2
        idx = scratch[o : o + _NSLOT]
        g = scratch[o + _NSLOT : o + 2 * _NSLOT]
        ot = scratch[o + 2 * _NSLOT : o + 3 * _NSLOT]
        sems = scratch[o + 3 * _NSLOT :]
        isem = sems[:_NSLOT]
        gsem = sems[_NSLOT : 2 * _NSLOT]
        osem = sems[2 * _NSLOT : 3 * _NSLOT]
        psem = sems[3 * _NSLOT : 4 * _NSLOT]

        wid = lax.axis_index("s") * info.num_cores + lax.axis_index("c")
        row0 = wid * rows_per_w

        @pl.when(lax.axis_index("s") == 0)
        def _():
            for c in range(st_n):
                pltpu.sync_copy(pos_hbm.at[:, :, pl.ds(c * _CH, _CH)], pt_tmp)
                pltpu.sync_copy(pt_tmp, pt_sh.at[c])

        plsc.subcore_barrier()

        def fire_idx(row, c, s):
            pltpu.async_copy(ids_hbm.at[row, pl.ds(c * _CH, _CH)], idx[s], isem[s])

        def wait_idx(s):
            pltpu.make_async_copy(ids_hbm.at[0, pl.ds(0, _CH)], idx[s], isem[s]).wait()

        def fire_gather(s):
            pltpu.async_copy(tok_hbm.at[idx[s]], g[s], gsem[s])

        def wait_gather(s):
            pltpu.make_async_copy(tok_hbm.at[pl.ds(0, _CH)], g[s], gsem[s]).wait()

        def fire_otinit(s):
            pltpu.async_copy(pt_sh.at[s], ot[s], psem[s])

        def wait_otinit(s):
            pltpu.make_async_copy(pt_sh.at[0], ot[s], psem[s]).wait()

        def fire_store(row, s):
            pltpu.async_copy(ot[s], out_hbm.at[row, :, s, :, :], osem[s])

        def wait_store(s):
            pltpu.make_async_copy(ot[s], out_hbm.at[0, :, 0, :, :], osem[s]).wait()

        for s in range(3):
            fire_idx(row0, s, s)
        fire_otinit(0)
        fire_otinit(1)
        wait_idx(0)
        fire_gather(0)
        wait_idx(1)
        fire_gather(1)

        iota = lax.iota(jnp.int32, _LANES)
        rowsv = [iota + (gg * _LANES) for gg in range(_CH // _LANES)]

        def group_body(kk, carry):
            row = row0 + kk
            for b in range(_NSLOT):
                k = kk * _NSLOT + b
                wait_gather(b)

                s3 = (b + 3) % _NSLOT

                @pl.when(k < nch - 3)
                def _():
                    fire_idx(row + (b + 3) // _NSLOT, (b + 3) % _NSLOT, s3)

                s2 = (b + 2) % _NSLOT

                @pl.when(k < nch - 2)
                def _():
                    @pl.when(k >= 2)
                    def _():
                        wait_store(s2)

                    fire_otinit(s2)
                    wait_idx(s2)
                    fire_gather(s2)

                wait_otinit(b)

                def add_body(et, c2):
                    for e8 in range(8):
                        colv = jnp.full((_LANES,), et * 8 + e8, jnp.int32)
                        for gg in range(_CH // _LANES):
                            v = plsc.load_gather(g[b], [rowsv[gg], colv])
                            plsc.addupdate(
                                ot[b].at[et, e8, pl.ds(gg * _LANES, _LANES)], v
                            )
                    return c2

                lax.fori_loop(0, 8, add_body, 0)
                fire_store(row, b)
            return carry

        lax.fori_loop(0, rows_per_w, group_body, 0)
        for s in range(_NSLOT):
            wait_store(s)

    return embed_kernel


def kernel(input_ids, token_table, position_table):
    batch, seq = input_ids.shape
    vocab, embed = token_table.shape
    fn = _build(batch, seq, embed, vocab)
    pos6 = position_table.T.reshape(8, 8, seq)
    out5 = fn(input_ids, token_table, pos6)
    return out5.transpose(0, 2, 4, 1, 3).reshape(batch, seq, embed)


# speedup vs baseline: 1.8661x; 1.8661x over previous
import functools

import jax
import jax.numpy as jnp
from jax import lax
from jax.experimental import pallas as pl
from jax.experimental.pallas import tpu as pltpu
from jax.experimental.pallas import tpu_sc as plsc

_LANES = 16
_CH = 128
_NSLOT = 4
_PAD = 129


@functools.cache
def _build(batch, seq, embed, vocab):
    info = plsc.get_sparse_core_info()
    nw = info.num_cores * info.num_subcores
    assert batch % nw == 0 and seq == _NSLOT * _CH and embed == 64
    rows_per_w = batch // nw
    nch = rows_per_w * _NSLOT
    st_n = seq // _CH
    ecols = embed // _LANES

    mesh = plsc.VectorSubcoreMesh(core_axis_name="c", subcore_axis_name="s")

    @functools.partial(
        pl.kernel,
        out_type=jax.ShapeDtypeStruct((batch, 8, st_n, 8, _CH), jnp.float32),
        mesh=mesh,
        compiler_params=pltpu.CompilerParams(
            use_tc_tiling_on_sc=False, needs_layout_passes=False
        ),
        scratch_types=(
            [pltpu.VMEM((seq, embed), jnp.float32)]
            + [pltpu.VMEM((_CH,), jnp.int32) for _ in range(_NSLOT)]
            + [pltpu.VMEM((_CH, 64), jnp.float32) for _ in range(_NSLOT)]
            + [pltpu.VMEM((8, 8, _PAD), jnp.float32) for _ in range(_NSLOT)]
            + [pltpu.SemaphoreType.DMA for _ in range(3 * _NSLOT)]
        ),
    )
    def embed_kernel(ids_hbm, tok_hbm, pos_hbm, out_hbm, pos_v, *scratch):
        idx = scratch[:_NSLOT]
        g = scratch[_NSLOT : 2 * _NSLOT]
        ot = scratch[2 * _NSLOT : 3 * _NSLOT]
        sems = scratch[3 * _NSLOT :]
        isem = sems[:_NSLOT]
        gsem = sems[_NSLOT : 2 * _NSLOT]
        osem = sems[2 * _NSLOT : 3 * _NSLOT]

        wid = lax.axis_index("s") * info.num_cores + lax.axis_index("c")
        row0 = wid * rows_per_w
        pltpu.sync_copy(pos_hbm, pos_v)

        def fire_idx(row, c, s):
            pltpu.async_copy(ids_hbm.at[row, pl.ds(c * _CH, _CH)], idx[s], isem[s])

        def wait_idx(s):
            pltpu.make_async_copy(ids_hbm.at[0, pl.ds(0, _CH)], idx[s], isem[s]).wait()

        def fire_gather(s):
            pltpu.async_copy(tok_hbm.at[idx[s]], g[s], gsem[s])

        def wait_gather(s):
            pltpu.make_async_copy(tok_hbm.at[pl.ds(0, _CH)], g[s], gsem[s]).wait()

        def fire_store(row, s):
            pltpu.async_copy(
                ot[s].at[:, :, pl.ds(0, _CH)], out_hbm.at[row, :, s, :, :], osem[s]
            )

        def wait_store(s):
            pltpu.make_async_copy(
                ot[s].at[:, :, pl.ds(0, _CH)], out_hbm.at[0, :, 0, :, :], osem[s]
            ).wait()

        for s in range(3):
            fire_idx(row0, s, s)
        wait_idx(0)
        fire_gather(0)
        wait_idx(1)
        fire_gather(1)

        iota = lax.iota(jnp.int32, _LANES)
        etv = [lax.shift_right_logical(iota + 16 * j, 3) for j in range(ecols)]
        e8v = [lax.bitwise_and(iota + 16 * j, 7) for j in range(ecols)]

        def group_body(kk, carry):
            row = row0 + kk
            for b in range(_NSLOT):
                k = kk * _NSLOT + b
                wait_gather(b)

                s3 = (b + 3) % _NSLOT

                @pl.when(k < nch - 3)
                def _():
                    fire_idx(row + (b + 3) // _NSLOT, (b + 3) % _NSLOT, s3)

                s2 = (b + 2) % _NSLOT

                @pl.when(k < nch - 2)
                def _():
                    @pl.when(k >= 2)
                    def _():
                        wait_store(s2)

                    wait_idx(s2)
                    fire_gather(s2)

                poff = b * _CH

                def add_body(ll, c2):
                    for l2 in range(4):
                        l = ll * 4 + l2
                        sv = jnp.full((_LANES,), l, jnp.int32)
                        for j in range(ecols):
                            sl = pl.ds(j * _LANES, _LANES)
                            v = g[b][l, sl] + pos_v[poff + l, sl]
                            plsc.store_scatter(ot[b], [etv[j], e8v[j], sv], v)
                    return c2

                lax.fori_loop(0, _CH // 4, add_body, 0)
                fire_store(row, b)
            return carry

        lax.fori_loop(0, rows_per_w, group_body, 0)
        for s in range(_NSLOT):
            wait_store(s)

    return embed_kernel


def kernel(input_ids, token_table, position_table):
    batch, seq = input_ids.shape
    vocab, embed = token_table.shape
    fn = _build(batch, seq, embed, vocab)
    out5 = fn(input_ids, token_table, position_table)
    return out5.transpose(0, 2, 4, 1, 3).reshape(batch, seq, embed)


# speedup vs baseline: 3.7065x; 1.9862x over previous
import functools

import jax
import jax.numpy as jnp
from jax import lax
from jax.experimental import pallas as pl
from jax.experimental.pallas import tpu as pltpu
from jax.experimental.pallas import tpu_sc as plsc

_LANES = 16
_CH = 128
_NSLOT = 4
_PAD = 129


@functools.cache
def _build(batch, seq, embed, vocab):
    info = plsc.get_sparse_core_info()
    nw = info.num_cores * info.num_subcores
    assert batch % nw == 0 and seq == _NSLOT * _CH and embed == 64
    rows_per_w = batch // nw
    nch = rows_per_w * _NSLOT
    st_n = seq // _CH
    ecols = embed // _LANES

    mesh = plsc.VectorSubcoreMesh(core_axis_name="c", subcore_axis_name="s")

    @functools.partial(
        pl.kernel,
        out_type=jax.ShapeDtypeStruct((batch, 8, st_n, 8, _CH), jnp.float32),
        mesh=mesh,
        compiler_params=pltpu.CompilerParams(
            use_tc_tiling_on_sc=False, needs_layout_passes=False
        ),
        scratch_types=(
            [pltpu.VMEM((seq, embed), jnp.float32)]
            + [pltpu.VMEM((_CH,), jnp.int32) for _ in range(_NSLOT)]
            + [pltpu.VMEM((_CH, 64), jnp.float32) for _ in range(_NSLOT)]
            + [pltpu.VMEM((8, 8, _PAD), jnp.float32) for _ in range(_NSLOT)]
            + [pltpu.SemaphoreType.DMA for _ in range(3 * _NSLOT)]
        ),
    )
    def embed_kernel(ids_hbm, tok_hbm, pos_hbm, out_hbm, pos_v, *scratch):
        idx = scratch[:_NSLOT]
        g = scratch[_NSLOT : 2 * _NSLOT]
        ot = scratch[2 * _NSLOT : 3 * _NSLOT]
        sems = scratch[3 * _NSLOT :]
        isem = sems[:_NSLOT]
        gsem = sems[_NSLOT : 2 * _NSLOT]
        osem = sems[2 * _NSLOT : 3 * _NSLOT]

        wid = lax.axis_index("s") * info.num_cores + lax.axis_index("c")
        row0 = wid * rows_per_w
        pltpu.sync_copy(pos_hbm, pos_v)

        def fire_idx(row, c, s):
            pltpu.async_copy(ids_hbm.at[row, pl.ds(c * _CH, _CH)], idx[s], isem[s])

        def wait_idx(s):
            pltpu.make_async_copy(ids_hbm.at[0, pl.ds(0, _CH)], idx[s], isem[s]).wait()

        def fire_gather(s):
            pltpu.async_copy(tok_hbm.at[idx[s]], g[s], gsem[s])

        def wait_gather(s):
            pltpu.make_async_copy(tok_hbm.at[pl.ds(0, _CH)], g[s], gsem[s]).wait()

        def fire_store(row, s):
            pltpu.async_copy(
                ot[s].at[:, :, pl.ds(0, _CH)], out_hbm.at[row, :, s, :, :], osem[s]
            )

        def wait_store(s):
            pltpu.make_async_copy(
                ot[s].at[:, :, pl.ds(0, _CH)], out_hbm.at[0, :, 0, :, :], osem[s]
            ).wait()

        for s in range(3):
            fire_idx(row0, s, s)
        wait_idx(0)
        fire_gather(0)
        wait_idx(1)
        fire_gather(1)

        iota = lax.iota(jnp.int32, _LANES)
        etv = [lax.shift_right_logical(iota + 16 * j, 3) for j in range(ecols)]
        e8v = [lax.bitwise_and(iota + 16 * j, 7) for j in range(ecols)]

        def group_body(kk, carry):
            row = row0 + kk
            for b in range(_NSLOT):
                k = kk * _NSLOT + b
                wait_gather(b)

                s3 = (b + 3) % _NSLOT

                @pl.when(k < nch - 3)
                def _():
                    fire_idx(row + (b + 3) // _NSLOT, (b + 3) % _NSLOT, s3)

                s2 = (b + 2) % _NSLOT

                @pl.when(k < nch - 2)
                def _():
                    @pl.when(k >= 2)
                    def _():
                        wait_store(s2)

                    wait_idx(s2)
                    fire_gather(s2)

                poff = b * _CH

                @plsc.parallel_loop(0, _CH, 1, unroll=8)
                def _(l):
                    sv = jnp.full((_LANES,), l, jnp.int32)
                    for j in range(ecols):
                        sl = pl.ds(j * _LANES, _LANES)
                        v = g[b][l, sl] + pos_v[poff + l, sl]
                        plsc.store_scatter(ot[b], [etv[j], e8v[j], sv], v)
                fire_store(row, b)
            return carry

        lax.fori_loop(0, rows_per_w, group_body, 0)
        for s in range(_NSLOT):
            wait_store(s)

    return embed_kernel


def kernel(input_ids, token_table, position_table):
    batch, seq = input_ids.shape
    vocab, embed = token_table.shape
    fn = _build(batch, seq, embed, vocab)
    out5 = fn(input_ids, token_table, position_table)
    return out5.transpose(0, 2, 4, 1, 3).reshape(batch, seq, embed)
